# baseline (speedup 1.0000x reference)
import functools

import jax
import jax.numpy as jnp
from jax import lax
from jax.experimental import pallas as pl
from jax.experimental.pallas import tpu as pltpu
from jax.experimental.pallas import tpu_sc as plsc

N = 10000
D = 128
HD = D // 2
E = 320000

NC = 2
NS = 16
NW = NC * NS
LANES = 16

NP = 10112
RPT = NP // NS
K = 128
CPT = 160
E_PAD = NS * CPT * K
EWT = CPT * K
NBUF = 4
NG = CPT // NBUF

EW_S = E_PAD // NW
CH_S = EW_S // K

_mesh = plsc.VectorSubcoreMesh(core_axis_name="c", subcore_axis_name="s")
_sc_params = pltpu.CompilerParams(needs_layout_passes=False,
                                  use_tc_tiling_on_sc=False)


@functools.partial(
    pl.kernel,
    out_type=jax.ShapeDtypeStruct((NW, NP), jnp.float32),
    mesh=_mesh,
    scratch_types=[
        pltpu.VMEM((EW_S,), jnp.int32),
        pltpu.VMEM((EW_S,), jnp.int32),
        pltpu.VMEM((NP,), jnp.float32),
        pltpu.VMEM((NP,), jnp.float32),
    ],
    compiler_params=_sc_params,
)
def _sc_scalar(table_hbm, src_hbm, dst_hbm, out_hbm,
               src_v, dst_v, table_v, acc_v):
    c = lax.axis_index("c")
    s = lax.axis_index("s")
    wid = s * NC + c
    z16 = jnp.zeros((LANES,), jnp.float32)

    def zero_blk(i, _):
        acc_v[pl.ds(i * LANES, LANES)] = z16
        return 0

    lax.fori_loop(0, NP // LANES, zero_blk, 0)
    pltpu.sync_copy(table_hbm, table_v)
    base = wid * EW_S
    pltpu.sync_copy(src_hbm.at[pl.ds(base, EW_S)], src_v)
    pltpu.sync_copy(dst_hbm.at[pl.ds(base, EW_S)], dst_v)

    def chunk(j, _):
        for i in range(K // LANES):
            o = j * K + i * LANES
            sv = src_v[pl.ds(o, LANES)]
            dv = dst_v[pl.ds(o, LANES)]
            vals = plsc.load_gather(table_v, [sv])
            plsc.addupdate_scatter(acc_v, [dv], vals)
        return 0

    lax.fori_loop(0, CH_S, chunk, 0)
    pltpu.sync_copy(acc_v, out_hbm.at[wid])


@functools.partial(
    pl.kernel,
    out_type=jax.ShapeDtypeStruct((NC, NP, HD), jnp.float32),
    mesh=_mesh,
    scratch_types=[
        [pltpu.VMEM((K,), jnp.int32) for _ in range(NBUF)],
        [pltpu.VMEM((K,), jnp.int32) for _ in range(NBUF)],
        [pltpu.VMEM((K, HD), jnp.float32) for _ in range(NBUF)],
        pltpu.VMEM_SHARED((NP, HD), jnp.float32),
        pltpu.VMEM_SHARED((NP, HD), jnp.float32),
        pltpu.SemaphoreType.DMA((NBUF,)),
        pltpu.SemaphoreType.DMA((NBUF,)),
        pltpu.SemaphoreType.DMA((NBUF,)),
        pltpu.SemaphoreType.DMA((NBUF,)),
    ],
    compiler_params=_sc_params,
)
def _sc_rows(y_hbm, src_hbm, dst_hbm, out_hbm,
             src_bufs, dst_bufs, row_bufs, y_sh, acc_sh, rsem, dsem, gsem,
             ssem):
    c = lax.axis_index("c")
    s = lax.axis_index("s")
    base = s * EWT
    rows = pl.ds(s * RPT, RPT)
    z16 = jnp.zeros((LANES,), jnp.float32)

    def zero_row(i, _):
        for j in range(HD // LANES):
            row_bufs[0][i, pl.ds(j * LANES, LANES)] = z16
        return 0

    lax.fori_loop(0, K, zero_row, 0)
    pltpu.sync_copy(y_hbm.at[c, rows], y_sh.at[rows])
    for k in range(RPT // K):
        pltpu.sync_copy(row_bufs[0],
                        acc_sh.at[pl.ds(s * RPT + k * K, K)])
    if RPT % K:
        pltpu.sync_copy(row_bufs[0].at[pl.ds(0, RPT % K)],
                        acc_sh.at[pl.ds(s * RPT + (RPT // K) * K, RPT % K)])
    plsc.subcore_barrier()

    def start_idx(cix, b):
        pltpu.async_copy(src_hbm.at[pl.ds(base + cix * K, K)],
                         src_bufs[b], rsem.at[b])
        pltpu.async_copy(dst_hbm.at[pl.ds(base + cix * K, K)],
                         dst_bufs[b], dsem.at[b])

    def start_gather(b):
        pltpu.async_copy(y_sh.at[src_bufs[b]], row_bufs[b], gsem.at[b])

    for b in range(NBUF):
        start_idx(b, b)

    def group(i, _):
        for b in range(NBUF):
            cix = i * NBUF + b
            pltpu.make_async_copy(src_hbm.at[pl.ds(base + cix * K, K)],
                                  src_bufs[b], rsem.at[b]).wait()
            start_gather(b)
        for b in range(NBUF):
            cix = i * NBUF + b
            pltpu.make_async_copy(dst_hbm.at[pl.ds(base + cix * K, K)],
                                  dst_bufs[b], dsem.at[b]).wait()
            pltpu.make_async_copy(y_sh.at[src_bufs[b]], row_bufs[b],
                                  gsem.at[b]).wait()
            pltpu.async_copy(row_bufs[b], acc_sh.at[dst_bufs[b]],
                             ssem.at[b], add=True)
        for b in range(NBUF):
            pltpu.make_async_copy(row_bufs[b], acc_sh.at[dst_bufs[b]],
                                  ssem.at[b]).wait()

            @pl.when(i < NG - 1)
            def _():
                start_idx((i + 1) * NBUF + b, b)
        return 0

    lax.fori_loop(0, NG, group, 0)
    plsc.subcore_barrier()
    pltpu.sync_copy(acc_sh.at[rows], out_hbm.at[c, rows])


def _col_sum(parts, ones_col):
    return lax.dot_general(parts, ones_col, (((0,), (0,)), ((), ())),
                           preferred_element_type=jnp.float32)


def _tc1_body(parts_ref, ones_ref, x_ref, w_ref, dinv_ref, y_ref):
    deg = _col_sum(parts_ref[...], ones_ref[...]) + 1.0
    dinv = lax.rsqrt(deg)
    dinv_ref[...] = dinv
    xw = jnp.dot(x_ref[...], w_ref[...], preferred_element_type=jnp.float32)
    y = xw * dinv
    y_ref[0] = y[:, :HD]
    y_ref[1] = y[:, HD:]


def _tc_mid_body(acc_ref, y_ref, dinv_ref, b_ref, w_ref, out_ref):
    dinv = dinv_ref[...]
    agg = jnp.concatenate([acc_ref[0] + y_ref[0], acc_ref[1] + y_ref[1]],
                          axis=1)
    h = jnp.maximum(dinv * agg + b_ref[...], 0.0)
    out = jnp.dot(h, w_ref[...], preferred_element_type=jnp.float32) * dinv
    if out_ref.ndim == 3:
        out_ref[0] = out[:, :HD]
        out_ref[1] = out[:, HD:]
    else:
        out_ref[...] = out


def _tc_out_body(parts_ref, ones_ref, y3_ref, dinv_ref, b3_ref, out_ref):
    a = _col_sum(parts_ref[...], ones_ref[...]) + y3_ref[...]
    out_ref[...] = jax.nn.sigmoid(dinv_ref[...] * a + b3_ref[...])


def kernel(x, edge_index, W1, b1, W2, b2, W3, b3):
    f32 = jnp.float32
    src = jnp.concatenate([edge_index[0],
                           jnp.zeros((E_PAD - E,), jnp.int32)])
    dst = jnp.concatenate([edge_index[1],
                           jnp.full((E_PAD - E,), N, jnp.int32)])
    xp = jnp.zeros((NP, D), f32).at[:N].set(x)

    ones_t = jnp.ones((NP,), f32)
    ones_col = jnp.ones((NW, 1), f32)

    deg_parts = _sc_scalar(ones_t, src, dst)

    dinv, y1 = pl.pallas_call(
        _tc1_body,
        out_shape=[jax.ShapeDtypeStruct((NP, 1), f32),
                   jax.ShapeDtypeStruct((NC, NP, HD), f32)],
    )(deg_parts, ones_col, xp, W1)

    acc1 = _sc_rows(y1, src, dst)
    y2 = pl.pallas_call(
        _tc_mid_body,
        out_shape=jax.ShapeDtypeStruct((NC, NP, HD), f32),
    )(acc1, y1, dinv, b1.reshape(1, D), W2)

    acc2 = _sc_rows(y2, src, dst)
    y3 = pl.pallas_call(
        _tc_mid_body,
        out_shape=jax.ShapeDtypeStruct((NP, 1), f32),
    )(acc2, y2, dinv, b2.reshape(1, D), W3)

    acc3_parts = _sc_scalar(y3.reshape(NP), src, dst)
    out = pl.pallas_call(
        _tc_out_body,
        out_shape=jax.ShapeDtypeStruct((NP, 1), f32),
    )(acc3_parts, ones_col, y3, dinv, b3.reshape(1, 1))
    return out[:N]


# speedup vs baseline: 1.0361x; 1.0361x over previous
import functools

import jax
import jax.numpy as jnp
from jax import lax
from jax.experimental import pallas as pl
from jax.experimental.pallas import tpu as pltpu
from jax.experimental.pallas import tpu_sc as plsc

N = 10000
D = 128
HD = D // 2
E = 320000

NC = 2
NS = 16
NW = NC * NS
LANES = 16

NP = 10112
RPT = NP // NS
K = 128
CPT = 160
E_PAD = NS * CPT * K
EWT = CPT * K
NBUF = 2
NG = CPT // NBUF

EW_S = E_PAD // NW
CH_S = EW_S // K

_mesh = plsc.VectorSubcoreMesh(core_axis_name="c", subcore_axis_name="s")
_sc_params = pltpu.CompilerParams(needs_layout_passes=False,
                                  use_tc_tiling_on_sc=False)


@functools.partial(
    pl.kernel,
    out_type=jax.ShapeDtypeStruct((NW, NP), jnp.float32),
    mesh=_mesh,
    scratch_types=[
        pltpu.VMEM((EW_S,), jnp.int32),
        pltpu.VMEM((EW_S,), jnp.int32),
        pltpu.VMEM((NP,), jnp.float32),
        pltpu.VMEM((NP,), jnp.float32),
    ],
    compiler_params=_sc_params,
)
def _sc_scalar(table_hbm, src_hbm, dst_hbm, out_hbm,
               src_v, dst_v, table_v, acc_v):
    c = lax.axis_index("c")
    s = lax.axis_index("s")
    wid = s * NC + c
    z16 = jnp.zeros((LANES,), jnp.float32)

    def zero_blk(i, _):
        acc_v[pl.ds(i * LANES, LANES)] = z16
        return 0

    lax.fori_loop(0, NP // LANES, zero_blk, 0)
    pltpu.sync_copy(table_hbm, table_v)
    base = wid * EW_S
    pltpu.sync_copy(src_hbm.at[pl.ds(base, EW_S)], src_v)
    pltpu.sync_copy(dst_hbm.at[pl.ds(base, EW_S)], dst_v)

    def chunk(j, _):
        for i in range(K // LANES):
            o = j * K + i * LANES
            sv = src_v[pl.ds(o, LANES)]
            dv = dst_v[pl.ds(o, LANES)]
            vals = plsc.load_gather(table_v, [sv])
            plsc.addupdate_scatter(acc_v, [dv], vals)
        return 0

    lax.fori_loop(0, CH_S, chunk, 0)
    pltpu.sync_copy(acc_v, out_hbm.at[wid])


@functools.partial(
    pl.kernel,
    out_type=jax.ShapeDtypeStruct((NC, NP, HD), jnp.float32),
    mesh=_mesh,
    scratch_types=[
        [pltpu.VMEM((K,), jnp.int32) for _ in range(NBUF)],
        [pltpu.VMEM((K,), jnp.int32) for _ in range(NBUF)],
        [pltpu.VMEM((K, HD), jnp.float32) for _ in range(NBUF)],
        pltpu.VMEM_SHARED((NP, HD), jnp.float32),
        pltpu.VMEM_SHARED((NP, HD), jnp.float32),
        pltpu.SemaphoreType.DMA((NBUF,)),
        pltpu.SemaphoreType.DMA((NBUF,)),
        pltpu.SemaphoreType.DMA((NBUF,)),
        pltpu.SemaphoreType.DMA((NBUF,)),
    ],
    compiler_params=_sc_params,
)
def _sc_rows(y_hbm, src_hbm, dst_hbm, out_hbm,
             src_bufs, dst_bufs, row_bufs, y_sh, acc_sh, rsem, dsem, gsem,
             ssem):
    c = lax.axis_index("c")
    s = lax.axis_index("s")
    base = s * EWT
    rows = pl.ds(s * RPT, RPT)
    z16 = jnp.zeros((LANES,), jnp.float32)

    def zero_row(i, _):
        for j in range(HD // LANES):
            row_bufs[0][i, pl.ds(j * LANES, LANES)] = z16
        return 0

    lax.fori_loop(0, K, zero_row, 0)
    pltpu.sync_copy(y_hbm.at[c, rows], y_sh.at[rows])
    for k in range(RPT // K):
        pltpu.sync_copy(row_bufs[0],
                        acc_sh.at[pl.ds(s * RPT + k * K, K)])
    if RPT % K:
        pltpu.sync_copy(row_bufs[0].at[pl.ds(0, RPT % K)],
                        acc_sh.at[pl.ds(s * RPT + (RPT // K) * K, RPT % K)])
    plsc.subcore_barrier()

    def start_idx(cix, b):
        pltpu.async_copy(src_hbm.at[pl.ds(base + cix * K, K)],
                         src_bufs[b], rsem.at[b])
        pltpu.async_copy(dst_hbm.at[pl.ds(base + cix * K, K)],
                         dst_bufs[b], dsem.at[b])

    def start_gather(b):
        pltpu.async_copy(y_sh.at[src_bufs[b]], row_bufs[b], gsem.at[b])

    for b in range(NBUF):
        start_idx(b, b)

    def group(i, _):
        for b in range(NBUF):
            cix = i * NBUF + b
            pltpu.make_async_copy(src_hbm.at[pl.ds(base + cix * K, K)],
                                  src_bufs[b], rsem.at[b]).wait()
            start_gather(b)
        for b in range(NBUF):
            cix = i * NBUF + b
            pltpu.make_async_copy(dst_hbm.at[pl.ds(base + cix * K, K)],
                                  dst_bufs[b], dsem.at[b]).wait()
            pltpu.make_async_copy(y_sh.at[src_bufs[b]], row_bufs[b],
                                  gsem.at[b]).wait()
            pltpu.async_copy(row_bufs[b], acc_sh.at[dst_bufs[b]],
                             ssem.at[b], add=True)
        for b in range(NBUF):
            pltpu.make_async_copy(row_bufs[b], acc_sh.at[dst_bufs[b]],
                                  ssem.at[b]).wait()

            @pl.when(i < NG - 1)
            def _():
                start_idx((i + 1) * NBUF + b, b)
        return 0

    lax.fori_loop(0, NG, group, 0)
    plsc.subcore_barrier()
    pltpu.sync_copy(acc_sh.at[rows], out_hbm.at[c, rows])


def _col_sum(parts, ones_col):
    return lax.dot_general(parts, ones_col, (((0,), (0,)), ((), ())),
                           preferred_element_type=jnp.float32)


def _tc1_body(parts_ref, ones_ref, x_ref, w_ref, dinv_ref, y_ref):
    deg = _col_sum(parts_ref[...], ones_ref[...]) + 1.0
    dinv = lax.rsqrt(deg)
    dinv_ref[...] = dinv
    xw = jnp.dot(x_ref[...], w_ref[...], preferred_element_type=jnp.float32)
    y = xw * dinv
    y_ref[0] = y[:, :HD]
    y_ref[1] = y[:, HD:]


def _tc_mid_body(acc_ref, y_ref, dinv_ref, b_ref, w_ref, out_ref):
    dinv = dinv_ref[...]
    agg = jnp.concatenate([acc_ref[0] + y_ref[0], acc_ref[1] + y_ref[1]],
                          axis=1)
    h = jnp.maximum(dinv * agg + b_ref[...], 0.0)
    out = jnp.dot(h, w_ref[...], preferred_element_type=jnp.float32) * dinv
    if out_ref.ndim == 3:
        out_ref[0] = out[:, :HD]
        out_ref[1] = out[:, HD:]
    else:
        out_ref[...] = out


def _tc_out_body(parts_ref, ones_ref, y3_ref, dinv_ref, b3_ref, out_ref):
    a = _col_sum(parts_ref[...], ones_ref[...]) + y3_ref[...]
    out_ref[...] = jax.nn.sigmoid(dinv_ref[...] * a + b3_ref[...])


def kernel(x, edge_index, W1, b1, W2, b2, W3, b3):
    f32 = jnp.float32
    src = jnp.concatenate([edge_index[0],
                           jnp.zeros((E_PAD - E,), jnp.int32)])
    dst = jnp.concatenate([edge_index[1],
                           jnp.full((E_PAD - E,), N, jnp.int32)])
    xp = jnp.zeros((NP, D), f32).at[:N].set(x)

    ones_t = jnp.ones((NP,), f32)
    ones_col = jnp.ones((NW, 1), f32)

    deg_parts = _sc_scalar(ones_t, src, dst)

    dinv, y1 = pl.pallas_call(
        _tc1_body,
        out_shape=[jax.ShapeDtypeStruct((NP, 1), f32),
                   jax.ShapeDtypeStruct((NC, NP, HD), f32)],
    )(deg_parts, ones_col, xp, W1)

    acc1 = _sc_rows(y1, src, dst)
    y2 = pl.pallas_call(
        _tc_mid_body,
        out_shape=jax.ShapeDtypeStruct((NC, NP, HD), f32),
    )(acc1, y1, dinv, b1.reshape(1, D), W2)

    acc2 = _sc_rows(y2, src, dst)
    y3 = pl.pallas_call(
        _tc_mid_body,
        out_shape=jax.ShapeDtypeStruct((NP, 1), f32),
    )(acc2, y2, dinv, b2.reshape(1, D), W3)

    acc3_parts = _sc_scalar(y3.reshape(NP), src, dst)
    out = pl.pallas_call(
        _tc_out_body,
        out_shape=jax.ShapeDtypeStruct((NP, 1), f32),
    )(acc3_parts, ones_col, y3, dinv, b3.reshape(1, 1))
    return out[:N]


# speedup vs baseline: 1.2100x; 1.1677x over previous
import functools

import jax
import jax.numpy as jnp
from jax import lax
from jax.experimental import pallas as pl
from jax.experimental.pallas import tpu as pltpu
from jax.experimental.pallas import tpu_sc as plsc

N = 10000
D = 128
HD = D // 2
E = 320000

NC = 2
NS = 16
NW = NC * NS
LANES = 16

NP = 10112
RPT = NP // NS
K = 128
CPT = 160
E_PAD = NS * CPT * K
EWT = CPT * K
NBUF = 2
NG = CPT // NBUF

EW_S = E_PAD // NW
CH_S = EW_S // K

_mesh = plsc.VectorSubcoreMesh(core_axis_name="c", subcore_axis_name="s")
_sc_params = pltpu.CompilerParams(needs_layout_passes=False,
                                  use_tc_tiling_on_sc=False)


@functools.partial(
    pl.kernel,
    out_type=jax.ShapeDtypeStruct((NW, NP), jnp.float32),
    mesh=_mesh,
    scratch_types=[
        pltpu.VMEM((EW_S,), jnp.int32),
        pltpu.VMEM((EW_S,), jnp.int32),
        pltpu.VMEM((NP,), jnp.float32),
        pltpu.VMEM((NP,), jnp.float32),
    ],
    compiler_params=_sc_params,
)
def _sc_scalar(table_hbm, src_hbm, dst_hbm, out_hbm,
               src_v, dst_v, table_v, acc_v):
    c = lax.axis_index("c")
    s = lax.axis_index("s")
    wid = s * NC + c
    z16 = jnp.zeros((LANES,), jnp.float32)

    def zero_blk(i, _):
        acc_v[pl.ds(i * LANES, LANES)] = z16
        return 0

    lax.fori_loop(0, NP // LANES, zero_blk, 0)
    pltpu.sync_copy(table_hbm, table_v)
    base = wid * EW_S
    pltpu.sync_copy(src_hbm.at[pl.ds(base, EW_S)], src_v)
    pltpu.sync_copy(dst_hbm.at[pl.ds(base, EW_S)], dst_v)

    def chunk(j, _):
        for i in range(K // LANES):
            o = j * K + i * LANES
            sv = src_v[pl.ds(o, LANES)]
            dv = dst_v[pl.ds(o, LANES)]
            vals = plsc.load_gather(table_v, [sv])
            plsc.addupdate_scatter(acc_v, [dv], vals)
        return 0

    lax.fori_loop(0, CH_S, chunk, 0)
    pltpu.sync_copy(acc_v, out_hbm.at[wid])


@functools.partial(
    pl.kernel,
    out_type=jax.ShapeDtypeStruct((NC, NP, HD), jnp.float32),
    mesh=_mesh,
    scratch_types=[
        pltpu.VMEM((EWT,), jnp.int32),
        [pltpu.VMEM((K,), jnp.int32) for _ in range(NBUF)],
        [pltpu.VMEM((K, HD), jnp.float32) for _ in range(NBUF)],
        pltpu.VMEM_SHARED((NP, HD), jnp.float32),
        pltpu.VMEM_SHARED((NP, HD), jnp.float32),
        pltpu.SemaphoreType.DMA((NBUF,)),
        pltpu.SemaphoreType.DMA((NBUF,)),
        pltpu.SemaphoreType.DMA((NBUF,)),
    ],
    compiler_params=_sc_params,
)
def _sc_rows(y_hbm, src_hbm, dst_hbm, out_hbm,
             src_v, dst_bufs, row_bufs, y_sh, acc_sh, dsem, gsem, ssem):
    c = lax.axis_index("c")
    s = lax.axis_index("s")
    base = s * EWT
    rows = pl.ds(s * RPT, RPT)
    z16 = jnp.zeros((LANES,), jnp.float32)

    def zero_row(i, _):
        for j in range(HD // LANES):
            row_bufs[0][i, pl.ds(j * LANES, LANES)] = z16
        return 0

    lax.fori_loop(0, K, zero_row, 0)
    pltpu.sync_copy(y_hbm.at[c, rows], y_sh.at[rows])
    for k in range(RPT // K):
        pltpu.sync_copy(row_bufs[0],
                        acc_sh.at[pl.ds(s * RPT + k * K, K)])
    if RPT % K:
        pltpu.sync_copy(row_bufs[0].at[pl.ds(0, RPT % K)],
                        acc_sh.at[pl.ds(s * RPT + (RPT // K) * K, RPT % K)])
    pltpu.sync_copy(src_hbm.at[pl.ds(base, EWT)], src_v)
    plsc.subcore_barrier()

    def start_chunk(cix, b):
        pltpu.async_copy(dst_hbm.at[pl.ds(base + cix * K, K)],
                         dst_bufs[b], dsem.at[b])
        pltpu.async_copy(y_sh.at[src_v.at[pl.ds(cix * K, K)]],
                         row_bufs[b], gsem.at[b])

    for b in range(NBUF):
        start_chunk(b, b)

    def group(i, _):
        for b in range(NBUF):
            cix = i * NBUF + b
            pltpu.make_async_copy(dst_hbm.at[pl.ds(base + cix * K, K)],
                                  dst_bufs[b], dsem.at[b]).wait()
            pltpu.make_async_copy(y_sh.at[src_v.at[pl.ds(cix * K, K)]],
                                  row_bufs[b], gsem.at[b]).wait()
            pltpu.async_copy(row_bufs[b], acc_sh.at[dst_bufs[b]],
                             ssem.at[b], add=True)
        for b in range(NBUF):
            pltpu.make_async_copy(row_bufs[b], acc_sh.at[dst_bufs[b]],
                                  ssem.at[b]).wait()

            @pl.when(i < NG - 1)
            def _():
                start_chunk((i + 1) * NBUF + b, b)
        return 0

    lax.fori_loop(0, NG, group, 0)
    plsc.subcore_barrier()
    pltpu.sync_copy(acc_sh.at[rows], out_hbm.at[c, rows])


def _col_sum(parts, ones_col):
    return lax.dot_general(parts, ones_col, (((0,), (0,)), ((), ())),
                           preferred_element_type=jnp.float32)


def _tc1_body(parts_ref, ones_ref, x_ref, w_ref, dinv_ref, y_ref):
    deg = _col_sum(parts_ref[...], ones_ref[...]) + 1.0
    dinv = lax.rsqrt(deg)
    dinv_ref[...] = dinv
    xw = jnp.dot(x_ref[...], w_ref[...], preferred_element_type=jnp.float32)
    y = xw * dinv
    y_ref[0] = y[:, :HD]
    y_ref[1] = y[:, HD:]


def _tc_mid_body(acc_ref, y_ref, dinv_ref, b_ref, w_ref, out_ref):
    dinv = dinv_ref[...]
    agg = jnp.concatenate([acc_ref[0] + y_ref[0], acc_ref[1] + y_ref[1]],
                          axis=1)
    h = jnp.maximum(dinv * agg + b_ref[...], 0.0)
    out = jnp.dot(h, w_ref[...], preferred_element_type=jnp.float32) * dinv
    if out_ref.ndim == 3:
        out_ref[0] = out[:, :HD]
        out_ref[1] = out[:, HD:]
    else:
        out_ref[...] = out


def _tc_out_body(parts_ref, ones_ref, y3_ref, dinv_ref, b3_ref, out_ref):
    a = _col_sum(parts_ref[...], ones_ref[...]) + y3_ref[...]
    out_ref[...] = jax.nn.sigmoid(dinv_ref[...] * a + b3_ref[...])


def kernel(x, edge_index, W1, b1, W2, b2, W3, b3):
    f32 = jnp.float32
    src = jnp.concatenate([edge_index[0],
                           jnp.zeros((E_PAD - E,), jnp.int32)])
    dst = jnp.concatenate([edge_index[1],
                           jnp.full((E_PAD - E,), N, jnp.int32)])
    xp = jnp.zeros((NP, D), f32).at[:N].set(x)

    ones_t = jnp.ones((NP,), f32)
    ones_col = jnp.ones((NW, 1), f32)

    deg_parts = _sc_scalar(ones_t, src, dst)

    dinv, y1 = pl.pallas_call(
        _tc1_body,
        out_shape=[jax.ShapeDtypeStruct((NP, 1), f32),
                   jax.ShapeDtypeStruct((NC, NP, HD), f32)],
    )(deg_parts, ones_col, xp, W1)

    acc1 = _sc_rows(y1, src, dst)
    y2 = pl.pallas_call(
        _tc_mid_body,
        out_shape=jax.ShapeDtypeStruct((NC, NP, HD), f32),
    )(acc1, y1, dinv, b1.reshape(1, D), W2)

    acc2 = _sc_rows(y2, src, dst)
    y3 = pl.pallas_call(
        _tc_mid_body,
        out_shape=jax.ShapeDtypeStruct((NP, 1), f32),
    )(acc2, y2, dinv, b2.reshape(1, D), W3)

    acc3_parts = _sc_scalar(y3.reshape(NP), src, dst)
    out = pl.pallas_call(
        _tc_out_body,
        out_shape=jax.ShapeDtypeStruct((NP, 1), f32),
    )(acc3_parts, ones_col, y3, dinv, b3.reshape(1, 1))
    return out[:N]
